# baseline (speedup 1.0000x reference)
import functools

import jax
import jax.numpy as jnp
from jax import lax
from jax.experimental import pallas as pl
from jax.experimental.pallas import tpu as pltpu
from jax.experimental.pallas import tpu_sc as plsc

_NC = 2
_NS = 16
_LANES = 16


def _sc_gather_body(F, V, B, spw, inT_hbm, tables_hbm, w_hbm, x26_hbm,
                    wv_hbm, in_v, idx_v, wv_all,
                    ebuf0, ebuf1, ebuf2, ebuf3, ebuf4, ebuf5,
                    esem0, esem1, esem2, esem3, esem4, esem5,
                    xsem0, xsem1, xsem2, xsem3, xsem4, xsem5, wsem, vsem):
    wid = lax.axis_index("s") * _NC + lax.axis_index("c")
    s0 = wid * spw
    npairs = spw * F

    pltpu.sync_copy(inT_hbm.at[pl.ds(wid * npairs, npairs)], in_v)

    def build_idx(f):
        for k in range(spw // _LANES):
            off = f * spw + k * _LANES
            idx_v[f, pl.ds(k * _LANES, _LANES)] = \
                in_v[pl.ds(off, _LANES)] + (f * V)

    ebufs = (ebuf0, ebuf1, ebuf2, ebuf3, ebuf4, ebuf5)
    esems = (esem0, esem1, esem2, esem3, esem4, esem5)
    xsems = (xsem0, xsem1, xsem2, xsem3, xsem4, xsem5)
    nslot = 6
    edesc = [None] * nslot
    xdesc = [None] * nslot
    wdesc = [None] * F
    vdesc = [None] * F

    for f in range(F):
        s = f % nslot
        build_idx(f)
        if f >= nslot:
            xdesc[s].wait()
        edesc[s] = pltpu.async_copy(tables_hbm.at[idx_v.at[f]],
                                    ebufs[s], esems[s])
        if f >= 8:
            wdesc[f - 8].wait()
        wdesc[f] = pltpu.async_copy(
            w_hbm.at[idx_v.at[f]], wv_all.at[pl.ds(f * spw, spw)], wsem)
        if f >= 1:
            p = (f - 1) % nslot
            edesc[p].wait()
            xdesc[p] = pltpu.async_copy(
                ebufs[p], x26_hbm.at[f - 1, pl.ds(s0, spw)], xsems[p])
    p = (F - 1) % nslot
    edesc[p].wait()
    xdesc[p] = pltpu.async_copy(
        ebufs[p], x26_hbm.at[F - 1, pl.ds(s0, spw)], xsems[p])
    for f in range(F - 8, F):
        wdesc[f].wait()
    for f in range(F):
        if f >= 8:
            vdesc[f - 8].wait()
        vdesc[f] = pltpu.async_copy(
            wv_all.at[pl.ds(f * spw, spw)],
            wv_hbm.at[pl.ds(f * B + s0, spw)], vsem)
    for f in range(F - 8, F):
        vdesc[f].wait()
    for s in range(nslot):
        xdesc[s].wait()


def _sc_gather(inputs_T_flat, tables_flat, w_flat, F, V, D):
    BF = inputs_T_flat.shape[0]
    B = BF // F
    spw = B // (_NC * _NS)
    mesh = plsc.VectorSubcoreMesh(core_axis_name="c", subcore_axis_name="s")
    kfn = pl.kernel(
        functools.partial(_sc_gather_body, F, V, B, spw),
        out_type=(
            jax.ShapeDtypeStruct((F, B, D), jnp.float32),
            jax.ShapeDtypeStruct((BF,), jnp.float32),
        ),
        mesh=mesh,
        compiler_params=pltpu.CompilerParams(use_tc_tiling_on_sc=False),
        scratch_types=[
            pltpu.VMEM((spw * F,), jnp.int32),
            pltpu.VMEM((F, spw), jnp.int32),
            pltpu.VMEM((spw * F,), jnp.float32),
            pltpu.VMEM((spw, D), jnp.float32),
            pltpu.VMEM((spw, D), jnp.float32),
            pltpu.VMEM((spw, D), jnp.float32),
            pltpu.VMEM((spw, D), jnp.float32),
            pltpu.VMEM((spw, D), jnp.float32),
            pltpu.VMEM((spw, D), jnp.float32),
        ] + [pltpu.SemaphoreType.DMA] * 14,
    )
    return kfn(inputs_T_flat, tables_flat, w_flat)


def _tc_dnn_body(F, x_ref, wv_ref, W1_ref, b1_ref, W2_ref, b2_ref,
                 W3_ref, b3_ref, Wf_ref, bf_ref, o_ref):
    bm = x_ref.shape[1]
    h1 = W1_ref.shape[1]
    acc = jnp.zeros((bm, h1), jnp.float32)
    for t in range(F // 2):
        xp = jnp.concatenate([x_ref[2 * t], x_ref[2 * t + 1]], axis=1)
        xp8 = (xp * (2.0 ** 8)).astype(jnp.float8_e4m3fn)
        wp = W1_ref[pl.ds(t * 256, 256), :]
        acc = acc + jnp.dot(xp8, wp, preferred_element_type=jnp.float32)
    h = jnp.maximum(acc * (2.0 ** -13) + b1_ref[...], 0.0).astype(jnp.bfloat16)
    h = jnp.maximum(
        jnp.dot(h, W2_ref[...], preferred_element_type=jnp.float32)
        + b2_ref[...], 0.0).astype(jnp.bfloat16)
    h = jnp.maximum(
        jnp.dot(h, W3_ref[...], preferred_element_type=jnp.float32)
        + b3_ref[...], 0.0)
    d = jnp.sum(h * Wf_ref[...], axis=1, keepdims=True) + bf_ref[0, 0]
    wide = jnp.sum(wv_ref[...], axis=0)[:, None]
    o_ref[...] = jax.nn.sigmoid(0.5 * wide + 0.5 * d)


def _tc_dnn(x26, wv2, W1, b1, W2, b2, W3, b3, Wf, bf):
    F, B, D = x26.shape
    DIN = W1.shape[0]
    H1, H2, H3 = W1.shape[1], W2.shape[1], W3.shape[1]
    BM = 256
    grid = (B // BM,)
    return pl.pallas_call(
        functools.partial(_tc_dnn_body, F),
        grid=grid,
        in_specs=[
            pl.BlockSpec((F, BM, D), lambda i: (0, i, 0)),
            pl.BlockSpec((F, BM), lambda i: (0, i)),
            pl.BlockSpec((DIN, H1), lambda i: (0, 0)),
            pl.BlockSpec((1, H1), lambda i: (0, 0)),
            pl.BlockSpec((H1, H2), lambda i: (0, 0)),
            pl.BlockSpec((1, H2), lambda i: (0, 0)),
            pl.BlockSpec((H2, H3), lambda i: (0, 0)),
            pl.BlockSpec((1, H3), lambda i: (0, 0)),
            pl.BlockSpec((1, H3), lambda i: (0, 0)),
            pl.BlockSpec((1, 1), lambda i: (0, 0)),
        ],
        out_specs=pl.BlockSpec((BM, 1), lambda i: (i, 0)),
        out_shape=jax.ShapeDtypeStruct((B, 1), jnp.float32),
    )(x26, wv2, W1, b1, W2, b2, W3, b3, Wf, bf)


def kernel(inputs, embed_tables, w_lin, W1, b1, W2, b2, W3, b3, Wf, bf):
    B, F = inputs.shape
    _, V, D = embed_tables.shape
    tables_flat = embed_tables.reshape(F * V, D)
    nw = _NC * _NS
    spw = B // nw
    inputs_T_flat = inputs.reshape(nw, spw, F).transpose(0, 2, 1).reshape(B * F)

    x26, wv = _sc_gather(inputs_T_flat, tables_flat, w_lin.reshape(F * V),
                         F, V, D)
    wv2 = wv.reshape(F, B)

    H1 = W1.shape[1]
    out = _tc_dnn(x26, wv2,
                  (W1 * (2.0 ** 5)).astype(jnp.float8_e4m3fn),
                  b1.reshape(1, H1),
                  W2.astype(jnp.bfloat16), b2.reshape(1, -1),
                  W3.astype(jnp.bfloat16), b3.reshape(1, -1),
                  Wf.reshape(1, -1), bf.reshape(1, 1))
    return out


# speedup vs baseline: 1.0466x; 1.0466x over previous
import functools

import jax
import jax.numpy as jnp
from jax import lax
from jax.experimental import pallas as pl
from jax.experimental.pallas import tpu as pltpu
from jax.experimental.pallas import tpu_sc as plsc

_NC = 2
_NS = 16
_LANES = 16


def _sc_gather_body(F, V, B, spw, inT_hbm, tables_hbm, w_hbm, x26_hbm,
                    wv_hbm, in_v, idx_v, wv_all,
                    ebuf0, ebuf1, ebuf2, ebuf3, ebuf4, ebuf5,
                    esem0, esem1, esem2, esem3, esem4, esem5,
                    xsem0, xsem1, xsem2, xsem3, xsem4, xsem5, wsem, vsem):
    wid = lax.axis_index("s") * _NC + lax.axis_index("c")
    s0 = wid * spw
    npairs = spw * F

    pltpu.sync_copy(inT_hbm.at[pl.ds(wid * npairs, npairs)], in_v)

    def build_idx(f):
        for k in range(spw // _LANES):
            off = f * spw + k * _LANES
            idx_v[f, pl.ds(k * _LANES, _LANES)] = \
                in_v[pl.ds(off, _LANES)] + (f * V)

    ebufs = (ebuf0, ebuf1, ebuf2, ebuf3, ebuf4, ebuf5)
    esems = (esem0, esem1, esem2, esem3, esem4, esem5)
    xsems = (xsem0, xsem1, xsem2, xsem3, xsem4, xsem5)
    nslot = 6
    edesc = [None] * nslot
    xdesc = [None] * nslot
    wdesc = [None] * F
    vdesc = [None] * F

    for f in range(F):
        s = f % nslot
        build_idx(f)
        if f >= nslot:
            xdesc[s].wait()
        edesc[s] = pltpu.async_copy(tables_hbm.at[idx_v.at[f]],
                                    ebufs[s], esems[s])
        if f >= 8:
            wdesc[f - 8].wait()
        wdesc[f] = pltpu.async_copy(
            w_hbm.at[idx_v.at[f]], wv_all.at[pl.ds(f * spw, spw)], wsem)
        if f >= 1:
            p = (f - 1) % nslot
            edesc[p].wait()
            xdesc[p] = pltpu.async_copy(
                ebufs[p], x26_hbm.at[f - 1, pl.ds(s0, spw)], xsems[p])
    p = (F - 1) % nslot
    edesc[p].wait()
    xdesc[p] = pltpu.async_copy(
        ebufs[p], x26_hbm.at[F - 1, pl.ds(s0, spw)], xsems[p])
    for f in range(F - 8, F):
        wdesc[f].wait()
    for f in range(F):
        if f >= 8:
            vdesc[f - 8].wait()
        vdesc[f] = pltpu.async_copy(
            wv_all.at[pl.ds(f * spw, spw)],
            wv_hbm.at[pl.ds(f * B + s0, spw)], vsem)
    for f in range(F - 8, F):
        vdesc[f].wait()
    for s in range(nslot):
        xdesc[s].wait()


def _sc_gather(inputs_T_flat, tables_flat, w_flat, F, V, D):
    BF = inputs_T_flat.shape[0]
    B = BF // F
    spw = B // (_NC * _NS)
    mesh = plsc.VectorSubcoreMesh(core_axis_name="c", subcore_axis_name="s")
    kfn = pl.kernel(
        functools.partial(_sc_gather_body, F, V, B, spw),
        out_type=(
            jax.ShapeDtypeStruct((F, B, D), jnp.float32),
            jax.ShapeDtypeStruct((BF,), jnp.float32),
        ),
        mesh=mesh,
        compiler_params=pltpu.CompilerParams(use_tc_tiling_on_sc=False),
        scratch_types=[
            pltpu.VMEM((spw * F,), jnp.int32),
            pltpu.VMEM((F, spw), jnp.int32),
            pltpu.VMEM((spw * F,), jnp.float32),
            pltpu.VMEM((spw, D), jnp.float32),
            pltpu.VMEM((spw, D), jnp.float32),
            pltpu.VMEM((spw, D), jnp.float32),
            pltpu.VMEM((spw, D), jnp.float32),
            pltpu.VMEM((spw, D), jnp.float32),
            pltpu.VMEM((spw, D), jnp.float32),
        ] + [pltpu.SemaphoreType.DMA] * 14,
    )
    return kfn(inputs_T_flat, tables_flat, w_flat)


def _tc_dnn_body(F, x_ref, wv_ref, W1_ref, b1_ref, W2_ref, b2_ref,
                 W3_ref, b3_ref, Wf_ref, bf_ref, o_ref):
    bm = x_ref.shape[1]
    h1 = W1_ref.shape[1]
    acc = jnp.zeros((bm, h1), jnp.float32)
    for t in range(F // 2):
        xp = jnp.concatenate([x_ref[2 * t], x_ref[2 * t + 1]], axis=1)
        xp8 = (xp * (2.0 ** 8)).astype(jnp.float8_e4m3fn)
        wp = W1_ref[pl.ds(t * 256, 256), :]
        acc = acc + jnp.dot(xp8, wp, preferred_element_type=jnp.float32)
    h = jnp.maximum(acc * (2.0 ** -13) + b1_ref[...], 0.0).astype(jnp.bfloat16)
    h = jnp.maximum(
        jnp.dot(h, W2_ref[...], preferred_element_type=jnp.float32)
        + b2_ref[...], 0.0).astype(jnp.bfloat16)
    h = jnp.maximum(
        jnp.dot(h, W3_ref[...], preferred_element_type=jnp.float32)
        + b3_ref[...], 0.0)
    d = jnp.sum(h * Wf_ref[...], axis=1, keepdims=True) + bf_ref[0, 0]
    wide = jnp.sum(wv_ref[...], axis=0)[:, None]
    o_ref[...] = jax.nn.sigmoid(0.5 * wide + 0.5 * d)


def _tc_dnn(x26, wv2, W1, b1, W2, b2, W3, b3, Wf, bf):
    F, B, D = x26.shape
    DIN = W1.shape[0]
    H1, H2, H3 = W1.shape[1], W2.shape[1], W3.shape[1]
    BM = 512
    grid = (B // BM,)
    return pl.pallas_call(
        functools.partial(_tc_dnn_body, F),
        grid=grid,
        in_specs=[
            pl.BlockSpec((F, BM, D), lambda i: (0, i, 0)),
            pl.BlockSpec((F, BM), lambda i: (0, i)),
            pl.BlockSpec((DIN, H1), lambda i: (0, 0)),
            pl.BlockSpec((1, H1), lambda i: (0, 0)),
            pl.BlockSpec((H1, H2), lambda i: (0, 0)),
            pl.BlockSpec((1, H2), lambda i: (0, 0)),
            pl.BlockSpec((H2, H3), lambda i: (0, 0)),
            pl.BlockSpec((1, H3), lambda i: (0, 0)),
            pl.BlockSpec((1, H3), lambda i: (0, 0)),
            pl.BlockSpec((1, 1), lambda i: (0, 0)),
        ],
        out_specs=pl.BlockSpec((BM, 1), lambda i: (i, 0)),
        out_shape=jax.ShapeDtypeStruct((B, 1), jnp.float32),
    )(x26, wv2, W1, b1, W2, b2, W3, b3, Wf, bf)


def kernel(inputs, embed_tables, w_lin, W1, b1, W2, b2, W3, b3, Wf, bf):
    B, F = inputs.shape
    _, V, D = embed_tables.shape
    tables_flat = embed_tables.reshape(F * V, D)
    nw = _NC * _NS
    spw = B // nw
    inputs_T_flat = inputs.reshape(nw, spw, F).transpose(0, 2, 1).reshape(B * F)

    x26, wv = _sc_gather(inputs_T_flat, tables_flat, w_lin.reshape(F * V),
                         F, V, D)
    wv2 = wv.reshape(F, B)

    H1 = W1.shape[1]
    out = _tc_dnn(x26, wv2,
                  (W1 * (2.0 ** 5)).astype(jnp.float8_e4m3fn),
                  b1.reshape(1, H1),
                  W2.astype(jnp.bfloat16), b2.reshape(1, -1),
                  W3.astype(jnp.bfloat16), b3.reshape(1, -1),
                  Wf.reshape(1, -1), bf.reshape(1, 1))
    return out
